# baseline (speedup 1.0000x reference)
import dataclasses
import functools

import jax
import jax.numpy as jnp
from jax import lax
from jax.experimental import pallas as pl
from jax.experimental.pallas import tpu as pltpu
from jax.experimental.pallas import tpu_sc as plsc

_B, _S, _H = 16, 4096, 128
_EPS = 1e-05
_RG = 16
_WPR = 2
_TPW = _S // _WPR
_K = 128
_NSUB = _TPW // _K
_L = 16


def _sc_embed_sum(state_flat, statep_flat, token_table, instruction_table,
                  argument_table):
    mesh = plsc.VectorSubcoreMesh(core_axis_name="c", subcore_axis_name="s")
    cp = pltpu.CompilerParams()
    if "needs_layout_passes" in pltpu.CompilerParams.__dataclass_fields__:
        cp = dataclasses.replace(cp, needs_layout_passes=False)

    @functools.partial(
        pl.kernel,
        mesh=mesh,
        compiler_params=cp,
        out_type=jax.ShapeDtypeStruct((_RG * _S, _H // 2), jnp.int32),
        scratch_types=[
            pltpu.VMEM((_TPW,), jnp.int32),
            pltpu.VMEM((_TPW,), jnp.int32),
            pltpu.VMEM((_TPW,), jnp.int32),
            pltpu.VMEM((_TPW,), jnp.int32),
            pltpu.VMEM((2, _K, _H), jnp.float32),
            pltpu.VMEM((2, 8, _H), jnp.float32),
            pltpu.VMEM((_K, _H), jnp.float32),
            pltpu.VMEM((2, _K + 8, _H), jnp.float32),
            pltpu.VMEM((2, _K, _H // 2), jnp.int32),
            pltpu.SMEM((_NSUB,), jnp.int32),
            pltpu.SMEM((_NSUB,), jnp.int32),
            pltpu.SMEM((_NSUB,), jnp.int32),
            pltpu.SemaphoreType.DMA,
            pltpu.SemaphoreType.DMA,
            pltpu.SemaphoreType.DMA,
            pltpu.SemaphoreType.DMA,
            pltpu.SemaphoreType.DMA,
            pltpu.SemaphoreType.DMA,
            pltpu.SemaphoreType.DMA,
            pltpu.SemaphoreType.DMA,
        ],
    )
    def body(state_hbm, statep_hbm, tok_hbm, ins_hbm, arg_hbm, out_hbm,
             sv, spv, iiv, aiv, tokb2, insb2, insS, argb2, outb2,
             ns_sm, ci_sm, ba_sm,
             st0, st1, si0, si1, sa0, sa1, so0, so1):
        w = lax.axis_index("s") * 2 + lax.axis_index("c")
        row = w // _WPR
        q = lax.rem(w, _WPR)
        row_base = row * _S
        base = row_base + q * _TPW
        local0 = q * _TPW

        iota = lax.iota(jnp.int32, 16)

        def _prescan(_):
            def quarter(k, carry):
                pltpu.sync_copy(
                    statep_hbm.at[pl.ds(row_base + k * _TPW, _TPW)], spv)

                def pre_body(g, carry2):
                    ci, cs = carry2
                    sp16 = spv[pl.ds(g * _L, _L)]
                    starts = sp16 == 0
                    ci = ci + jnp.sum(
                        jnp.where(starts, 1, 0).astype(jnp.int32))
                    pos16 = iota + (k * _TPW + g * _L)
                    cs = jnp.maximum(
                        cs, jnp.max(jnp.where(starts, pos16, 0)))
                    return ci, cs

                return lax.fori_loop(0, _TPW // _L, pre_body, carry)

            return lax.fori_loop(0, q, quarter,
                                 (jnp.int32(0), jnp.int32(0)))

        ci0, cs0 = lax.cond(q > 0, _prescan,
                            lambda _: (jnp.int32(0), jnp.int32(0)), 0)

        pltpu.sync_copy(state_hbm.at[pl.ds(base, _TPW)], sv)
        pltpu.sync_copy(statep_hbm.at[pl.ds(base, _TPW)], spv)

        def idx_body(j, carry):
            ci, cs = carry
            ci_sm[j] = ci
            ba_sm[j] = local0 + j * _K - cs
            ci_in = ci
            for g in range(8):
                off = j * _K + g * _L
                sp16 = spv[pl.ds(off, _L)]
                starts = sp16 == 0
                s16 = jnp.where(starts, 1, 0).astype(jnp.int32)
                inst16 = plsc.cumsum(s16) + ci
                ci = jnp.max(inst16)
                pos16 = iota + (local0 + off)
                mpos = jnp.where(starts, pos16, 0)
                seg16 = jnp.maximum(plsc.cummax(mpos), cs)
                cs = jnp.max(seg16)
                iiv[pl.ds(off, _L)] = inst16
                aiv[pl.ds(off, _L)] = pos16 - seg16
            ns_sm[j] = ci - ci_in
            return ci, cs

        lax.fori_loop(0, _NSUB, idx_body, (ci0, cs0))

        sem_tok = (st0, st1)
        sem_ins = (si0, si1)
        sem_arg = (sa0, sa1)
        sem_out = (so0, so1)

        def _bases(j):
            ci_j = ci_sm[j]
            ci8 = pl.multiple_of(
                jnp.minimum(ci_j - lax.rem(ci_j, 8), 4096 - 8), 8)
            ba_j = ba_sm[j]
            ba8 = pl.multiple_of(
                jnp.minimum(ba_j - lax.rem(ba_j, 8), 4096 - (_K + 8)), 8)
            return ci_j, ci8, ba_j, ba8

        def issue(j, p):
            sl = pl.ds(j * _K, _K)
            pltpu.async_copy(tok_hbm.at[sv.at[sl]], tokb2.at[p], sem_tok[p])
            ns = ns_sm[j]
            _, ci8, _, ba8 = _bases(j)

            def fast(_):
                pltpu.async_copy(ins_hbm.at[pl.ds(ci8, 8)],
                                 insb2.at[p], sem_ins[p])
                pltpu.async_copy(arg_hbm.at[pl.ds(ba8, _K + 8)],
                                 argb2.at[p], sem_arg[p])
                return 0

            def slow(_):
                pltpu.async_copy(arg_hbm.at[aiv.at[sl]],
                                 argb2.at[p].at[pl.ds(0, _K)], sem_arg[p])
                return 0

            lax.cond(ns == 0, fast, slow, 0)

        def consume(j, p):
            sl = pl.ds(j * _K, _K)
            ns = ns_sm[j]
            ci_j, ci8, ba_j, ba8 = _bases(j)
            io = ci_j - ci8
            ao = ba_j - ba8

            def dfast(_):
                pltpu.make_async_copy(ins_hbm.at[pl.ds(0, 8)],
                                      insb2.at[p], sem_ins[p]).wait()
                pltpu.make_async_copy(arg_hbm.at[pl.ds(0, _K + 8)],
                                      argb2.at[p], sem_arg[p]).wait()
                pltpu.make_async_copy(tok_hbm.at[sv.at[sl]], tokb2.at[p],
                                      sem_tok[p]).wait()
                ivals = [insb2[p, io, pl.ds(hh * _L, _L)]
                         for hh in range(_H // _L)]

                def add_fast(t, c):
                    ng = _H // _L
                    tv = [tokb2[p, t, pl.ds(hh * _L, _L)] for hh in range(ng)]
                    av = [argb2[p, t + ao, pl.ds(hh * _L, _L)]
                          for hh in range(ng)]
                    sm = [tv[hh] + ivals[hh] + av[hh] for hh in range(ng)]
                    for h2 in range(ng // 2):
                        w32 = plsc.pack(sm[2 * h2], sm[2 * h2 + 1],
                                        format=plsc.PackFormat.INTERLEAVED)
                        outb2[p, t, pl.ds(h2 * _L, _L)] = plsc.bitcast(
                            w32, jnp.int32)
                    return c

                lax.fori_loop(0, _K, add_fast, 0)
                return 0

            def dslow(_):
                cpi = pltpu.async_copy(ins_hbm.at[iiv.at[sl]], insS,
                                       sem_ins[p])
                cpi.wait()
                pltpu.make_async_copy(arg_hbm.at[pl.ds(0, _K)],
                                      argb2.at[p].at[pl.ds(0, _K)],
                                      sem_arg[p]).wait()
                pltpu.make_async_copy(tok_hbm.at[sv.at[sl]], tokb2.at[p],
                                      sem_tok[p]).wait()

                def add_body(t, c):
                    ng = _H // _L
                    tv = [tokb2[p, t, pl.ds(hh * _L, _L)] for hh in range(ng)]
                    iv = [insS[t, pl.ds(hh * _L, _L)] for hh in range(ng)]
                    av = [argb2[p, t, pl.ds(hh * _L, _L)] for hh in range(ng)]
                    sm = [tv[hh] + iv[hh] + av[hh] for hh in range(ng)]
                    for h2 in range(ng // 2):
                        w32 = plsc.pack(sm[2 * h2], sm[2 * h2 + 1],
                                        format=plsc.PackFormat.INTERLEAVED)
                        outb2[p, t, pl.ds(h2 * _L, _L)] = plsc.bitcast(
                            w32, jnp.int32)
                    return c

                lax.fori_loop(0, _K, add_body, 0)
                return 0

            lax.cond(ns == 0, dfast, dslow, 0)
            pltpu.async_copy(outb2.at[p],
                             out_hbm.at[pl.ds(base + j * _K, _K)], sem_out[p])

        def wait_out(j, p):
            pltpu.make_async_copy(outb2.at[p],
                                  out_hbm.at[pl.ds(base + j * _K, _K)],
                                  sem_out[p]).wait()

        issue(0, 0)
        for j in range(_NSUB):
            p = j & 1
            if j + 1 < _NSUB:
                if j >= 1:
                    wait_out(j - 1, 1 - p)
                issue(j + 1, 1 - p)
            consume(j, p)
        wait_out(_NSUB - 2, 0)
        wait_out(_NSUB - 1, 1)

    return body(state_flat, statep_flat, token_table, instruction_table,
                argument_table)


def _ln_body(x_ref, w_ref, b_ref, o_ref):
    x = x_ref[...].astype(jnp.float32)
    mu = jnp.mean(x, axis=-1, keepdims=True)
    xc = x - mu
    var = jnp.mean(xc * xc, axis=-1, keepdims=True)
    inv = lax.rsqrt(var + _EPS)
    o_ref[...] = xc * inv * w_ref[...] + b_ref[...]


def _layernorm(summed, ln_weight, ln_bias):
    blk = 8192
    n = summed.shape[0]
    return pl.pallas_call(
        _ln_body,
        grid=(n // blk,),
        in_specs=[
            pl.BlockSpec((blk, _H), lambda i: (i, 0)),
            pl.BlockSpec((1, _H), lambda i: (0, 0)),
            pl.BlockSpec((1, _H), lambda i: (0, 0)),
        ],
        out_specs=pl.BlockSpec((blk, _H), lambda i: (i, 0)),
        out_shape=jax.ShapeDtypeStruct((n, _H), jnp.float32),
    )(summed, ln_weight.reshape(1, _H), ln_bias.reshape(1, _H))


def _preperm(t):
    n = t.shape[0]
    return (t.reshape(n, _H // 32, 16, 2).transpose(0, 1, 3, 2)
            .reshape(n, _H))


def kernel(state, token_table, instruction_table, argument_table, ln_weight,
           ln_bias):
    state = state.astype(jnp.int32)
    statep = jnp.roll(state, 1, axis=-1).at[:, 0].set(1)
    groups = []
    for g in range(_B // _RG):
        rows = slice(g * _RG, (g + 1) * _RG)
        words = _sc_embed_sum(state[rows].reshape(-1),
                              statep[rows].reshape(-1),
                              _preperm(token_table),
                              _preperm(instruction_table),
                              _preperm(argument_table))
        summed = lax.bitcast_convert_type(
            words, jnp.bfloat16).reshape(_RG * _S, _H)
        groups.append(_layernorm(summed, ln_weight, ln_bias))
    out = jnp.concatenate(groups, axis=0)
    return out.reshape(_B, _S, _H)


# speedup vs baseline: 4.0983x; 4.0983x over previous
import dataclasses
import functools

import jax
import jax.numpy as jnp
from jax import lax
from jax.experimental import pallas as pl
from jax.experimental.pallas import tpu as pltpu
from jax.experimental.pallas import tpu_sc as plsc

_B, _S, _H = 16, 4096, 128
_EPS = 1e-05
_RG = 16
_WPR = 2
_TPW = _S // _WPR
_K = 128
_NSUB = _TPW // _K
_L = 16


def _sc_embed_sum(state_flat, statep_flat, token_table, instruction_table,
                  argument_table):
    mesh = plsc.VectorSubcoreMesh(core_axis_name="c", subcore_axis_name="s")
    cp = pltpu.CompilerParams()
    if "needs_layout_passes" in pltpu.CompilerParams.__dataclass_fields__:
        cp = dataclasses.replace(cp, needs_layout_passes=False)

    @functools.partial(
        pl.kernel,
        mesh=mesh,
        compiler_params=cp,
        out_type=jax.ShapeDtypeStruct((_RG * _S, _H), jnp.float32),
        scratch_types=[
            pltpu.VMEM((_TPW,), jnp.int32),
            pltpu.VMEM((_TPW,), jnp.int32),
            pltpu.VMEM((_TPW,), jnp.int32),
            pltpu.VMEM((_TPW,), jnp.int32),
            pltpu.VMEM((2, _K, _H), jnp.float32),
            pltpu.VMEM((2, _K, _H), jnp.float32),
            pltpu.VMEM((2, _K + 8, _H), jnp.float32),
            pltpu.SMEM((_NSUB,), jnp.int32),
            pltpu.SMEM((_NSUB,), jnp.int32),
            pltpu.SMEM((_NSUB,), jnp.int32),
            pltpu.SemaphoreType.DMA,
            pltpu.SemaphoreType.DMA,
            pltpu.SemaphoreType.DMA,
            pltpu.SemaphoreType.DMA,
            pltpu.SemaphoreType.DMA,
            pltpu.SemaphoreType.DMA,
            pltpu.SemaphoreType.DMA,
            pltpu.SemaphoreType.DMA,
        ],
    )
    def body(state_hbm, statep_hbm, tok_hbm, ins_hbm, arg_hbm, out_hbm,
             sv, spv, iiv, aiv, tokb2, insb2, argb2, ns_sm, ci_sm, ba_sm,
             st0, st1, si0, si1, sa0, sa1, so0, so1):
        w = lax.axis_index("s") * 2 + lax.axis_index("c")
        row = w // _WPR
        q = lax.rem(w, _WPR)
        row_base = row * _S
        base = row_base + q * _TPW
        local0 = q * _TPW

        iota = lax.iota(jnp.int32, 16)

        def _prescan(_):
            def quarter(k, carry):
                pltpu.sync_copy(
                    statep_hbm.at[pl.ds(row_base + k * _TPW, _TPW)], spv)

                def pre_body(g, carry2):
                    ci, cs = carry2
                    sp16 = spv[pl.ds(g * _L, _L)]
                    starts = sp16 == 0
                    ci = ci + jnp.sum(
                        jnp.where(starts, 1, 0).astype(jnp.int32))
                    pos16 = iota + (k * _TPW + g * _L)
                    cs = jnp.maximum(
                        cs, jnp.max(jnp.where(starts, pos16, 0)))
                    return ci, cs

                return lax.fori_loop(0, _TPW // _L, pre_body, carry)

            return lax.fori_loop(0, q, quarter,
                                 (jnp.int32(0), jnp.int32(0)))

        ci0, cs0 = lax.cond(q > 0, _prescan,
                            lambda _: (jnp.int32(0), jnp.int32(0)), 0)

        pltpu.sync_copy(state_hbm.at[pl.ds(base, _TPW)], sv)
        pltpu.sync_copy(statep_hbm.at[pl.ds(base, _TPW)], spv)

        def idx_body(j, carry):
            ci, cs = carry
            ci_sm[j] = ci
            ba_sm[j] = local0 + j * _K - cs
            ci_in = ci
            for g in range(8):
                off = j * _K + g * _L
                sp16 = spv[pl.ds(off, _L)]
                starts = sp16 == 0
                s16 = jnp.where(starts, 1, 0).astype(jnp.int32)
                inst16 = plsc.cumsum(s16) + ci
                ci = jnp.max(inst16)
                pos16 = iota + (local0 + off)
                mpos = jnp.where(starts, pos16, 0)
                seg16 = jnp.maximum(plsc.cummax(mpos), cs)
                cs = jnp.max(seg16)
                iiv[pl.ds(off, _L)] = inst16
                aiv[pl.ds(off, _L)] = pos16 - seg16
            ns_sm[j] = ci - ci_in
            return ci, cs

        lax.fori_loop(0, _NSUB, idx_body, (ci0, cs0))

        sem_tok = (st0, st1)
        sem_ins = (si0, si1)
        sem_arg = (sa0, sa1)
        sem_out = (so0, so1)

        def _bases(j):
            ci_j = ci_sm[j]
            ci8 = pl.multiple_of(
                jnp.minimum(ci_j - lax.rem(ci_j, 8), 4096 - 8), 8)
            ba_j = ba_sm[j]
            ba8 = pl.multiple_of(
                jnp.minimum(ba_j - lax.rem(ba_j, 8), 4096 - (_K + 8)), 8)
            return ci_j, ci8, ba_j, ba8

        def issue(j, p):
            sl = pl.ds(j * _K, _K)
            pltpu.async_copy(tok_hbm.at[sv.at[sl]], tokb2.at[p], sem_tok[p])
            ns = ns_sm[j]
            _, ci8, _, ba8 = _bases(j)

            def fast(_):
                pltpu.async_copy(ins_hbm.at[pl.ds(ci8, 8)],
                                 insb2.at[p].at[pl.ds(0, 8)], sem_ins[p])
                pltpu.async_copy(arg_hbm.at[pl.ds(ba8, _K + 8)],
                                 argb2.at[p], sem_arg[p])
                return 0

            def slow(_):
                pltpu.async_copy(ins_hbm.at[iiv.at[sl]],
                                 insb2.at[p].at[pl.ds(0, _K)], sem_ins[p])
                pltpu.async_copy(arg_hbm.at[aiv.at[sl]],
                                 argb2.at[p].at[pl.ds(0, _K)], sem_arg[p])
                return 0

            lax.cond(ns == 0, fast, slow, 0)

        def consume(j, p):
            sl = pl.ds(j * _K, _K)
            ns = ns_sm[j]
            ci_j, ci8, ba_j, ba8 = _bases(j)
            io = ci_j - ci8
            ao = ba_j - ba8

            def dfast(_):
                pltpu.make_async_copy(ins_hbm.at[pl.ds(0, 8)],
                                      insb2.at[p].at[pl.ds(0, 8)],
                                      sem_ins[p]).wait()
                pltpu.make_async_copy(arg_hbm.at[pl.ds(0, _K + 8)],
                                      argb2.at[p], sem_arg[p]).wait()
                pltpu.make_async_copy(tok_hbm.at[sv.at[sl]], tokb2.at[p],
                                      sem_tok[p]).wait()
                ivals = [insb2[p, io, pl.ds(hh * _L, _L)]
                         for hh in range(_H // _L)]

                def add_fast(t, c):
                    ng = _H // _L
                    tv = [tokb2[p, t, pl.ds(hh * _L, _L)] for hh in range(ng)]
                    av = [argb2[p, t + ao, pl.ds(hh * _L, _L)]
                          for hh in range(ng)]
                    sm = [tv[hh] + ivals[hh] + av[hh] for hh in range(ng)]
                    for hh in range(ng):
                        tokb2[p, t, pl.ds(hh * _L, _L)] = sm[hh]
                    return c

                lax.fori_loop(0, _K, add_fast, 0)
                return 0

            def dslow(_):
                pltpu.make_async_copy(ins_hbm.at[pl.ds(0, _K)],
                                      insb2.at[p].at[pl.ds(0, _K)],
                                      sem_ins[p]).wait()
                pltpu.make_async_copy(arg_hbm.at[pl.ds(0, _K)],
                                      argb2.at[p].at[pl.ds(0, _K)],
                                      sem_arg[p]).wait()
                pltpu.make_async_copy(tok_hbm.at[sv.at[sl]], tokb2.at[p],
                                      sem_tok[p]).wait()

                def add_body(t, c):
                    ng = _H // _L
                    tv = [tokb2[p, t, pl.ds(hh * _L, _L)] for hh in range(ng)]
                    iv = [insb2[p, t, pl.ds(hh * _L, _L)] for hh in range(ng)]
                    av = [argb2[p, t, pl.ds(hh * _L, _L)] for hh in range(ng)]
                    sm = [tv[hh] + iv[hh] + av[hh] for hh in range(ng)]
                    for hh in range(ng):
                        tokb2[p, t, pl.ds(hh * _L, _L)] = sm[hh]
                    return c

                lax.fori_loop(0, _K, add_body, 0)
                return 0

            lax.cond(ns == 0, dfast, dslow, 0)
            pltpu.async_copy(tokb2.at[p],
                             out_hbm.at[pl.ds(base + j * _K, _K)], sem_out[p])

        def wait_out(j, p):
            pltpu.make_async_copy(tokb2.at[p],
                                  out_hbm.at[pl.ds(base + j * _K, _K)],
                                  sem_out[p]).wait()

        issue(0, 0)
        for j in range(_NSUB):
            p = j & 1
            if j + 1 < _NSUB:
                if j >= 1:
                    wait_out(j - 1, 1 - p)
                issue(j + 1, 1 - p)
            consume(j, p)
        wait_out(_NSUB - 2, 0)
        wait_out(_NSUB - 1, 1)

    return body(state_flat, statep_flat, token_table, instruction_table,
                argument_table)


def _ln_body(x_ref, w_ref, b_ref, o_ref):
    x = x_ref[...]
    mu = jnp.mean(x, axis=-1, keepdims=True)
    xc = x - mu
    var = jnp.mean(xc * xc, axis=-1, keepdims=True)
    inv = lax.rsqrt(var + _EPS)
    o_ref[...] = xc * inv * w_ref[...] + b_ref[...]


def _layernorm(summed, ln_weight, ln_bias):
    blk = 8192
    n = summed.shape[0]
    return pl.pallas_call(
        _ln_body,
        grid=(n // blk,),
        in_specs=[
            pl.BlockSpec((blk, _H), lambda i: (i, 0)),
            pl.BlockSpec((1, _H), lambda i: (0, 0)),
            pl.BlockSpec((1, _H), lambda i: (0, 0)),
        ],
        out_specs=pl.BlockSpec((blk, _H), lambda i: (i, 0)),
        out_shape=jax.ShapeDtypeStruct((n, _H), jnp.float32),
    )(summed, ln_weight.reshape(1, _H), ln_bias.reshape(1, _H))


def kernel(state, token_table, instruction_table, argument_table, ln_weight,
           ln_bias):
    state = state.astype(jnp.int32)
    statep = jnp.roll(state, 1, axis=-1).at[:, 0].set(1)
    groups = []
    for g in range(_B // _RG):
        rows = slice(g * _RG, (g + 1) * _RG)
        summed = _sc_embed_sum(state[rows].reshape(-1),
                               statep[rows].reshape(-1),
                               token_table, instruction_table,
                               argument_table)
        groups.append(_layernorm(summed, ln_weight, ln_bias))
    out = jnp.concatenate(groups, axis=0)
    return out.reshape(_B, _S, _H)
